# Initial kernel scaffold; baseline (speedup 1.0000x reference)
import jax
import jax.numpy as jnp
from jax.experimental import pallas as pl


def kernel(x, edge_index, W1, b1, W2, b2, Wc, bc):
    raise NotImplementedError("write your pallas kernel here")


# speedup vs baseline: 12.8548x; 12.8548x over previous
import functools

import jax
import jax.numpy as jnp
from jax import lax
from jax.experimental import pallas as pl
from jax.experimental.pallas import tpu as pltpu
from jax.experimental.pallas import tpu_sc as plsc

NC = 2
NS = 16
NW = NC * NS
L = 16
C = 128


def _sc_deg(dst_pad, np_pad, epw):
    nps = np_pad // NS
    nchunk = epw // C
    mesh = plsc.VectorSubcoreMesh(core_axis_name="c", subcore_axis_name="s", num_cores=NC, num_subcores=NS)

    @functools.partial(
        pl.kernel,
        out_type=jax.ShapeDtypeStruct((NC * np_pad,), jnp.float32),
        mesh=mesh,
        scratch_types=[
            pltpu.VMEM((C,), jnp.int32),
            pltpu.VMEM((C,), jnp.float32),
            pltpu.VMEM((nps,), jnp.float32),
            pltpu.VMEM_SHARED((np_pad,), jnp.float32),
        ],
    )
    def k(dst_hbm, out_hbm, didx, ones, zbuf, acc):
        c = lax.axis_index("c")
        s = lax.axis_index("s")
        wid = s * NC + c
        base = wid * epw

        one = jnp.full((L,), 1.0, jnp.float32)
        zero = jnp.zeros((L,), jnp.float32)
        for i in range(C // L):
            ones[pl.ds(i * L, L)] = one
        for i in range(nps // L):
            zbuf[pl.ds(i * L, L)] = zero
        pltpu.sync_copy(zbuf, acc.at[pl.ds(s * nps, nps)])
        plsc.subcore_barrier()

        def body(i, carry):
            pltpu.sync_copy(dst_hbm.at[pl.ds(base + i * C, C)], didx)
            pltpu.sync_copy(ones, acc.at[didx], add=True)
            return carry

        lax.fori_loop(0, nchunk, body, 0)
        plsc.subcore_barrier()
        pltpu.sync_copy(acc.at[pl.ds(s * nps, nps)], zbuf)
        pltpu.sync_copy(zbuf, out_hbm.at[pl.ds(c * np_pad + s * nps, nps)])

    return k(dst_pad)


def _sc_prop(srcf, dstf, u_flat, np_pad, epw, f):
    nf = (np_pad // NS) * f
    nchunk = epw // C
    mesh = plsc.VectorSubcoreMesh(core_axis_name="c", subcore_axis_name="s", num_cores=NC, num_subcores=NS)

    @functools.partial(
        pl.kernel,
        out_type=jax.ShapeDtypeStruct((NC * np_pad * f,), jnp.float32),
        mesh=mesh,
        scratch_types=[
            pltpu.VMEM((C,), jnp.int32),
            pltpu.VMEM((C,), jnp.int32),
            pltpu.VMEM((C,), jnp.float32),
            pltpu.VMEM((nf,), jnp.float32),
            pltpu.VMEM_SHARED((np_pad * f,), jnp.float32),
            pltpu.SemaphoreType.DMA,
        ],
    )
    def k(src_hbm, dst_hbm, u_hbm, out_hbm, sidx, didx, vals, zbuf, acc, sem):
        c = lax.axis_index("c")
        s = lax.axis_index("s")
        wid = s * NC + c
        base = wid * epw

        pltpu.sync_copy(u_hbm.at[pl.ds(s * nf, nf)], zbuf)
        pltpu.sync_copy(zbuf, acc.at[pl.ds(s * nf, nf)])
        plsc.subcore_barrier()

        one = jnp.full((L,), 1, jnp.int32)

        def body(i, carry):
            pltpu.sync_copy(src_hbm.at[pl.ds(base + i * C, C)], sidx)
            pltpu.sync_copy(dst_hbm.at[pl.ds(base + i * C, C)], didx)
            for j in range(f):
                if j > 0:
                    for t in range(C // L):
                        sidx[pl.ds(t * L, L)] = sidx[pl.ds(t * L, L)] + one
                        didx[pl.ds(t * L, L)] = didx[pl.ds(t * L, L)] + one
                pltpu.async_copy(u_hbm.at[sidx], vals, sem).wait()
                pltpu.sync_copy(vals, acc.at[didx], add=True)
            return carry

        lax.fori_loop(0, nchunk, body, 0)
        plsc.subcore_barrier()
        pltpu.sync_copy(acc.at[pl.ds(s * nf, nf)], zbuf)
        pltpu.sync_copy(zbuf, out_hbm.at[pl.ds(c * np_pad * f + s * nf, nf)])

    return k(srcf, dstf, u_flat)


def _tc_call(body, out_shapes, *args):
    return pl.pallas_call(
        body,
        out_shape=[jax.ShapeDtypeStruct(s, jnp.float32) for s in out_shapes],
    )(*args)


def kernel(x, edge_index, W1, b1, W2, b2, Wc, bc):
    n, d = x.shape
    e = edge_index.shape[1]
    h1 = W1.shape[1]
    h2 = W2.shape[1]

    align = NS * L
    np_pad = ((n + align - 1) // align) * align
    if np_pad == n:
        np_pad = n + align
    e_pad = ((e + NW * C - 1) // (NW * C)) * (NW * C)
    epw = e_pad // NW

    src = edge_index[0]
    dst = edge_index[1]
    pad = e_pad - e
    src_pad = jnp.concatenate([src, jnp.zeros((pad,), jnp.int32)])
    dst_pad = jnp.concatenate([dst, jnp.full((pad,), np_pad - 1, jnp.int32)])

    degp = _sc_deg(dst_pad, np_pad, epw)
    degp2 = degp.reshape((NC * np_pad, 1))

    def tc1(x_ref, w1_ref, degp_ref, u1_ref, dinv_ref):
        deg = degp_ref[0:np_pad, :] + degp_ref[np_pad:2 * np_pad, :] + 1.0
        dinv = lax.rsqrt(deg)
        dinv_ref[...] = dinv
        hw = jnp.dot(x_ref[...], w1_ref[...], preferred_element_type=jnp.float32)
        u1_ref[0:n, :] = hw * dinv[0:n, :]
        u1_ref[n:np_pad, :] = jnp.zeros((np_pad - n, h1), jnp.float32)

    u1, dinv = _tc_call(tc1, [(np_pad, h1), (np_pad, 1)], x, W1, degp2)

    accp1 = _sc_prop(src_pad * h1, dst_pad * h1, u1.reshape(np_pad * h1),
                     np_pad, epw, h1).reshape(NC * np_pad, h1)

    def tc2(accp_ref, u1_ref, dinv_ref, b1_ref, w2_ref, u2_ref):
        acc = (accp_ref[0:np_pad, :] + accp_ref[np_pad:2 * np_pad, :]
               - u1_ref[...])
        hh = jnp.maximum(dinv_ref[0:n, :] * acc[0:n, :] + b1_ref[...], 0.0)
        u2_ref[0:n, :] = jnp.dot(hh, w2_ref[...],
                                 preferred_element_type=jnp.float32) * dinv_ref[0:n, :]
        u2_ref[n:np_pad, :] = jnp.zeros((np_pad - n, h2), jnp.float32)

    u2, = _tc_call(tc2, [(np_pad, h2)], accp1, u1, dinv, b1.reshape(1, h1), W2)

    accp2 = _sc_prop(src_pad * h2, dst_pad * h2, u2.reshape(np_pad * h2),
                     np_pad, epw, h2).reshape(NC * np_pad, h2)

    def tc3(accp_ref, u2_ref, dinv_ref, b2_ref, wc_ref, bc_ref, out_ref, h_ref):
        acc = (accp_ref[0:np_pad, :] + accp_ref[np_pad:2 * np_pad, :]
               - u2_ref[...])
        hh = jnp.tanh(dinv_ref[0:n, :] * acc[0:n, :] + b2_ref[...])
        h_ref[...] = hh
        z = jnp.dot(hh, wc_ref[...], preferred_element_type=jnp.float32) + bc_ref[...]
        out_ref[...] = jax.nn.sigmoid(z)

    out, h = _tc_call(tc3, [(n, 1), (n, h2)], accp2, u2, dinv,
                      b2.reshape(1, h2), Wc, bc.reshape(1, 1))
    return (out, h)


# speedup vs baseline: 21.5100x; 1.6733x over previous
import functools

import jax
import jax.numpy as jnp
from jax import lax
from jax.experimental import pallas as pl
from jax.experimental.pallas import tpu as pltpu
from jax.experimental.pallas import tpu_sc as plsc

NC = 2
NS = 16
NW = NC * NS
L = 16
C = 128


def _sc_deg(dst_pad, np_pad, epw):
    nps = np_pad // NS
    nchunk = epw // C
    mesh = plsc.VectorSubcoreMesh(core_axis_name="c", subcore_axis_name="s", num_cores=NC, num_subcores=NS)

    @functools.partial(
        pl.kernel,
        out_type=jax.ShapeDtypeStruct((NC * np_pad,), jnp.float32),
        mesh=mesh,
        scratch_types=[
            pltpu.VMEM((C,), jnp.int32),
            pltpu.VMEM((C,), jnp.float32),
            pltpu.VMEM((nps,), jnp.float32),
            pltpu.VMEM_SHARED((np_pad,), jnp.float32),
        ],
    )
    def k(dst_hbm, out_hbm, didx, ones, zbuf, acc):
        c = lax.axis_index("c")
        s = lax.axis_index("s")
        wid = s * NC + c
        base = wid * epw

        one = jnp.full((L,), 1.0, jnp.float32)
        zero = jnp.zeros((L,), jnp.float32)
        for i in range(C // L):
            ones[pl.ds(i * L, L)] = one
        for i in range(nps // L):
            zbuf[pl.ds(i * L, L)] = zero
        pltpu.sync_copy(zbuf, acc.at[pl.ds(s * nps, nps)])
        plsc.subcore_barrier()

        def body(i, carry):
            pltpu.sync_copy(dst_hbm.at[pl.ds(base + i * C, C)], didx)
            pltpu.sync_copy(ones, acc.at[didx], add=True)
            return carry

        lax.fori_loop(0, nchunk, body, 0)
        plsc.subcore_barrier()
        pltpu.sync_copy(acc.at[pl.ds(s * nps, nps)], zbuf)
        pltpu.sync_copy(zbuf, out_hbm.at[pl.ds(c * np_pad + s * nps, nps)])

    return k(dst_pad)


def _sc_prop(srcf, dstf, u_flat, np_pad, epw, f):
    nf = (np_pad // NS) * f
    nchunk = epw // C
    mesh = plsc.VectorSubcoreMesh(core_axis_name="c", subcore_axis_name="s", num_cores=NC, num_subcores=NS)

    @functools.partial(
        pl.kernel,
        out_type=jax.ShapeDtypeStruct((NC * np_pad * f,), jnp.float32),
        mesh=mesh,
        scratch_types=[
            pltpu.VMEM((C,), jnp.int32),
            pltpu.VMEM((C,), jnp.int32),
            pltpu.VMEM((C,), jnp.float32),
            pltpu.VMEM((nf,), jnp.float32),
            pltpu.VMEM_SHARED((np_pad * f,), jnp.float32),
            pltpu.SemaphoreType.DMA,
        ],
    )
    def k(src_hbm, dst_hbm, u_hbm, out_hbm, sidx, didx, vals, zbuf, acc, sem):
        c = lax.axis_index("c")
        s = lax.axis_index("s")
        wid = s * NC + c
        base = wid * epw

        pltpu.sync_copy(u_hbm.at[pl.ds(s * nf, nf)], zbuf)
        pltpu.sync_copy(zbuf, acc.at[pl.ds(s * nf, nf)])
        plsc.subcore_barrier()

        one = jnp.full((L,), 1, jnp.int32)

        def body(i, carry):
            pltpu.sync_copy(src_hbm.at[pl.ds(base + i * C, C)], sidx)
            pltpu.sync_copy(dst_hbm.at[pl.ds(base + i * C, C)], didx)
            for j in range(f):
                if j > 0:
                    for t in range(C // L):
                        sidx[pl.ds(t * L, L)] = sidx[pl.ds(t * L, L)] + one
                        didx[pl.ds(t * L, L)] = didx[pl.ds(t * L, L)] + one
                pltpu.async_copy(u_hbm.at[sidx], vals, sem).wait()
                pltpu.sync_copy(vals, acc.at[didx], add=True)
            return carry

        lax.fori_loop(0, nchunk, body, 0)
        plsc.subcore_barrier()
        pltpu.sync_copy(acc.at[pl.ds(s * nf, nf)], zbuf)
        pltpu.sync_copy(zbuf, out_hbm.at[pl.ds(c * np_pad * f + s * nf, nf)])

    return k(srcf, dstf, u_flat)


def _sc_prop16(src_pad, dst_pad, u16, np_pad, epw):
    nps = np_pad // NS
    nchunk = epw // C
    mesh = plsc.VectorSubcoreMesh(core_axis_name="c", subcore_axis_name="s", num_cores=NC, num_subcores=NS)

    @functools.partial(
        pl.kernel,
        out_type=jax.ShapeDtypeStruct((NC * np_pad, 16), jnp.float32),
        mesh=mesh,
        scratch_types=[
            pltpu.VMEM((C,), jnp.int32),
            pltpu.VMEM((C,), jnp.int32),
            pltpu.VMEM((C, 16), jnp.float32),
            pltpu.VMEM((nps, 16), jnp.float32),
            pltpu.VMEM_SHARED((np_pad, 16), jnp.float32),
            pltpu.SemaphoreType.DMA,
        ],
        compiler_params=pltpu.CompilerParams(use_tc_tiling_on_sc=False),
    )
    def k(src_hbm, dst_hbm, u_hbm, out_hbm, sidx, didx, rows, zbuf, acc, sem):
        c = lax.axis_index("c")
        s = lax.axis_index("s")
        wid = s * NC + c
        base = wid * epw

        pltpu.sync_copy(u_hbm.at[pl.ds(s * nps, nps)], zbuf)
        pltpu.sync_copy(zbuf, acc.at[pl.ds(s * nps, nps)])
        plsc.subcore_barrier()

        def body(i, carry):
            pltpu.sync_copy(src_hbm.at[pl.ds(base + i * C, C)], sidx)
            pltpu.sync_copy(dst_hbm.at[pl.ds(base + i * C, C)], didx)
            pltpu.async_copy(u_hbm.at[sidx], rows, sem).wait()
            pltpu.sync_copy(rows, acc.at[didx], add=True)
            return carry

        lax.fori_loop(0, nchunk, body, 0)
        plsc.subcore_barrier()
        pltpu.sync_copy(acc.at[pl.ds(s * nps, nps)], zbuf)
        pltpu.sync_copy(zbuf, out_hbm.at[pl.ds(c * np_pad + s * nps, nps)])

    return k(src_pad, dst_pad, u16)


def _tc_call(body, out_shapes, *args):
    return pl.pallas_call(
        body,
        out_shape=[jax.ShapeDtypeStruct(s, jnp.float32) for s in out_shapes],
    )(*args)


def kernel(x, edge_index, W1, b1, W2, b2, Wc, bc):
    n, d = x.shape
    e = edge_index.shape[1]
    h1 = W1.shape[1]
    h2 = W2.shape[1]

    align = NS * L
    np_pad = ((n + align - 1) // align) * align
    if np_pad == n:
        np_pad = n + align
    e_pad = ((e + NW * C - 1) // (NW * C)) * (NW * C)
    epw = e_pad // NW

    src = edge_index[0]
    dst = edge_index[1]
    pad = e_pad - e
    src_pad = jnp.concatenate([src, jnp.zeros((pad,), jnp.int32)])
    dst_pad = jnp.concatenate([dst, jnp.full((pad,), np_pad - 1, jnp.int32)])

    degp = _sc_deg(dst_pad, np_pad, epw)
    degp2 = degp.reshape((NC * np_pad, 1))

    def tc1(x_ref, w1_ref, degp_ref, u1_ref, dinv_ref):
        deg = degp_ref[0:np_pad, :] + degp_ref[np_pad:2 * np_pad, :] + 1.0
        dinv = lax.rsqrt(deg)
        dinv_ref[...] = dinv
        hw = jnp.dot(x_ref[...], w1_ref[...], preferred_element_type=jnp.float32)
        u1_ref[0:n, 0:h1] = hw * dinv[0:n, :]
        u1_ref[n:np_pad, 0:h1] = jnp.zeros((np_pad - n, h1), jnp.float32)
        u1_ref[:, h1:16] = jnp.zeros((np_pad, 16 - h1), jnp.float32)

    u1, dinv = _tc_call(tc1, [(np_pad, 16), (np_pad, 1)], x, W1, degp2)

    accp1 = _sc_prop16(src_pad, dst_pad, u1, np_pad, epw)

    def tc2(accp_ref, u1_ref, dinv_ref, b1_ref, w2_ref, u2_ref):
        acc = (accp_ref[0:np_pad, 0:h1] + accp_ref[np_pad:2 * np_pad, 0:h1]
               - u1_ref[:, 0:h1])
        hh = jnp.maximum(dinv_ref[0:n, :] * acc[0:n, :] + b1_ref[...], 0.0)
        u2_ref[0:n, 0:h2] = jnp.dot(hh, w2_ref[...],
                                    preferred_element_type=jnp.float32) * dinv_ref[0:n, :]
        u2_ref[n:np_pad, 0:h2] = jnp.zeros((np_pad - n, h2), jnp.float32)
        u2_ref[:, h2:16] = jnp.zeros((np_pad, 16 - h2), jnp.float32)

    u2, = _tc_call(tc2, [(np_pad, 16)], accp1, u1, dinv, b1.reshape(1, h1), W2)

    accp2 = _sc_prop16(src_pad, dst_pad, u2, np_pad, epw)

    def tc3(accp_ref, u2_ref, dinv_ref, b2_ref, wc_ref, bc_ref, out_ref, h_ref):
        acc = (accp_ref[0:np_pad, 0:h2] + accp_ref[np_pad:2 * np_pad, 0:h2]
               - u2_ref[:, 0:h2])
        hh = jnp.tanh(dinv_ref[0:n, :] * acc[0:n, :] + b2_ref[...])
        h_ref[...] = hh
        z = jnp.dot(hh, wc_ref[...], preferred_element_type=jnp.float32) + bc_ref[...]
        out_ref[...] = jax.nn.sigmoid(z)

    out, h = _tc_call(tc3, [(n, 1), (n, h2)], accp2, u2, dinv,
                      b2.reshape(1, h2), Wc, bc.reshape(1, 1))
    return (out, h)


# speedup vs baseline: 22.0087x; 1.0232x over previous
import functools

import jax
import jax.numpy as jnp
from jax import lax
from jax.experimental import pallas as pl
from jax.experimental.pallas import tpu as pltpu
from jax.experimental.pallas import tpu_sc as plsc

NC = 2
NS = 16
NW = NC * NS
L = 16
C = 128


def _sc_deg(dst_pad, np_pad, epw):
    nps = np_pad // NS
    nchunk = epw // C
    mesh = plsc.VectorSubcoreMesh(core_axis_name="c", subcore_axis_name="s", num_cores=NC, num_subcores=NS)

    @functools.partial(
        pl.kernel,
        out_type=jax.ShapeDtypeStruct((NC * np_pad,), jnp.float32),
        mesh=mesh,
        scratch_types=[
            pltpu.VMEM((C,), jnp.int32),
            pltpu.VMEM((C,), jnp.float32),
            pltpu.VMEM((nps,), jnp.float32),
            pltpu.VMEM_SHARED((np_pad,), jnp.float32),
        ],
    )
    def k(dst_hbm, out_hbm, didx, ones, zbuf, acc):
        c = lax.axis_index("c")
        s = lax.axis_index("s")
        wid = s * NC + c
        base = wid * epw

        one = jnp.full((L,), 1.0, jnp.float32)
        zero = jnp.zeros((L,), jnp.float32)
        for i in range(C // L):
            ones[pl.ds(i * L, L)] = one
        for i in range(nps // L):
            zbuf[pl.ds(i * L, L)] = zero
        pltpu.sync_copy(zbuf, acc.at[pl.ds(s * nps, nps)])
        plsc.subcore_barrier()

        def body(i, carry):
            pltpu.sync_copy(dst_hbm.at[pl.ds(base + i * C, C)], didx)
            pltpu.sync_copy(ones, acc.at[didx], add=True)
            return carry

        lax.fori_loop(0, nchunk, body, 0)
        plsc.subcore_barrier()
        pltpu.sync_copy(acc.at[pl.ds(s * nps, nps)], zbuf)
        pltpu.sync_copy(zbuf, out_hbm.at[pl.ds(c * np_pad + s * nps, nps)])

    return k(dst_pad)


def _sc_prop(srcf, dstf, u_flat, np_pad, epw, f):
    nf = (np_pad // NS) * f
    nchunk = epw // C
    mesh = plsc.VectorSubcoreMesh(core_axis_name="c", subcore_axis_name="s", num_cores=NC, num_subcores=NS)

    @functools.partial(
        pl.kernel,
        out_type=jax.ShapeDtypeStruct((NC * np_pad * f,), jnp.float32),
        mesh=mesh,
        scratch_types=[
            pltpu.VMEM((C,), jnp.int32),
            pltpu.VMEM((C,), jnp.int32),
            pltpu.VMEM((C,), jnp.float32),
            pltpu.VMEM((nf,), jnp.float32),
            pltpu.VMEM_SHARED((np_pad * f,), jnp.float32),
            pltpu.SemaphoreType.DMA,
        ],
    )
    def k(src_hbm, dst_hbm, u_hbm, out_hbm, sidx, didx, vals, zbuf, acc, sem):
        c = lax.axis_index("c")
        s = lax.axis_index("s")
        wid = s * NC + c
        base = wid * epw

        pltpu.sync_copy(u_hbm.at[pl.ds(s * nf, nf)], zbuf)
        pltpu.sync_copy(zbuf, acc.at[pl.ds(s * nf, nf)])
        plsc.subcore_barrier()

        one = jnp.full((L,), 1, jnp.int32)

        def body(i, carry):
            pltpu.sync_copy(src_hbm.at[pl.ds(base + i * C, C)], sidx)
            pltpu.sync_copy(dst_hbm.at[pl.ds(base + i * C, C)], didx)
            for j in range(f):
                if j > 0:
                    for t in range(C // L):
                        sidx[pl.ds(t * L, L)] = sidx[pl.ds(t * L, L)] + one
                        didx[pl.ds(t * L, L)] = didx[pl.ds(t * L, L)] + one
                pltpu.async_copy(u_hbm.at[sidx], vals, sem).wait()
                pltpu.sync_copy(vals, acc.at[didx], add=True)
            return carry

        lax.fori_loop(0, nchunk, body, 0)
        plsc.subcore_barrier()
        pltpu.sync_copy(acc.at[pl.ds(s * nf, nf)], zbuf)
        pltpu.sync_copy(zbuf, out_hbm.at[pl.ds(c * np_pad * f + s * nf, nf)])

    return k(srcf, dstf, u_flat)


def _sc_prop16(src_pad, dst_pad, u16, np_pad, epw):
    nps = np_pad // NS
    nchunk = epw // C
    mesh = plsc.VectorSubcoreMesh(core_axis_name="c", subcore_axis_name="s", num_cores=NC, num_subcores=NS)

    e_pad = epw * NW

    @functools.partial(
        pl.kernel,
        out_type=jax.ShapeDtypeStruct((NC * np_pad, 16), jnp.float32),
        mesh=mesh,
        scratch_types=[
            pltpu.VMEM((C,), jnp.int32),
            pltpu.VMEM((C,), jnp.int32),
            pltpu.VMEM((C,), jnp.int32),
            pltpu.VMEM((C,), jnp.int32),
            pltpu.VMEM((C, 16), jnp.float32),
            pltpu.VMEM((C, 16), jnp.float32),
            pltpu.VMEM((nps, 16), jnp.float32),
            pltpu.VMEM_SHARED((np_pad, 16), jnp.float32),
            pltpu.SemaphoreType.DMA,
            pltpu.SemaphoreType.DMA,
            pltpu.SemaphoreType.DMA,
            pltpu.SemaphoreType.DMA,
        ],
        compiler_params=pltpu.CompilerParams(use_tc_tiling_on_sc=False),
    )
    def k(src_hbm, dst_hbm, u_hbm, out_hbm,
          sidx0, sidx1, didx0, didx1, rows0, rows1, zbuf, acc,
          gsem0, gsem1, ssem0, ssem1):
        c = lax.axis_index("c")
        s = lax.axis_index("s")
        wid = s * NC + c
        base = wid * epw
        bufs = ((sidx0, didx0, rows0, gsem0, ssem0),
                (sidx1, didx1, rows1, gsem1, ssem1))

        pltpu.sync_copy(u_hbm.at[pl.ds(s * nps, nps)], zbuf)
        pltpu.sync_copy(zbuf, acc.at[pl.ds(s * nps, nps)])
        plsc.subcore_barrier()

        (sidx0, didx0, rows0, gsem0, _s0), (sidx1, didx1, rows1, gsem1, _s1) = bufs

        def body(i, carry):
            off0 = base + 2 * i * C
            pltpu.sync_copy(src_hbm.at[pl.ds(off0, C)], sidx0)
            pltpu.sync_copy(dst_hbm.at[pl.ds(off0, C)], didx0)
            g0 = pltpu.async_copy(u_hbm.at[sidx0], rows0, gsem0)
            pltpu.sync_copy(src_hbm.at[pl.ds(off0 + C, C)], sidx1)
            pltpu.sync_copy(dst_hbm.at[pl.ds(off0 + C, C)], didx1)
            g0.wait()
            g1 = pltpu.async_copy(u_hbm.at[sidx1], rows1, gsem1)
            g1.wait()
            pltpu.sync_copy(rows0, acc.at[didx0], add=True)
            pltpu.sync_copy(rows1, acc.at[didx1], add=True)
            return carry

        lax.fori_loop(0, nchunk // 2, body, 0)
        plsc.subcore_barrier()
        pltpu.sync_copy(acc.at[pl.ds(s * nps, nps)], zbuf)
        pltpu.sync_copy(zbuf, out_hbm.at[pl.ds(c * np_pad + s * nps, nps)])

    return k(src_pad, dst_pad, u16)


def _tc_call(body, out_shapes, *args):
    return pl.pallas_call(
        body,
        out_shape=[jax.ShapeDtypeStruct(s, jnp.float32) for s in out_shapes],
    )(*args)


def kernel(x, edge_index, W1, b1, W2, b2, Wc, bc):
    n, d = x.shape
    e = edge_index.shape[1]
    h1 = W1.shape[1]
    h2 = W2.shape[1]

    align = NS * L
    np_pad = ((n + align - 1) // (align)) * align
    while np_pad - n < C:
        np_pad += align
    e_pad = ((e + 2 * NW * C - 1) // (2 * NW * C)) * (2 * NW * C)
    epw = e_pad // NW

    src = edge_index[0]
    dst = edge_index[1]
    pad = e_pad - e
    src_pad = jnp.concatenate([src, jnp.zeros((pad,), jnp.int32)])
    dst_pad = jnp.concatenate([dst, jnp.full((pad,), np_pad - 1, jnp.int32)])

    degp = _sc_deg(dst_pad, np_pad, epw)
    degp2 = degp.reshape((NC * np_pad, 1))

    def tc1(x_ref, w1_ref, degp_ref, u1_ref, dinv_ref):
        deg = degp_ref[0:np_pad, :] + degp_ref[np_pad:2 * np_pad, :] + 1.0
        dinv = lax.rsqrt(deg)
        dinv_ref[...] = dinv
        hw = jnp.dot(x_ref[...], w1_ref[...], preferred_element_type=jnp.float32)
        u1_ref[0:n, 0:h1] = hw * dinv[0:n, :]
        u1_ref[n:np_pad, 0:h1] = jnp.zeros((np_pad - n, h1), jnp.float32)
        u1_ref[:, h1:16] = jnp.zeros((np_pad, 16 - h1), jnp.float32)

    u1, dinv = _tc_call(tc1, [(np_pad, 16), (np_pad, 1)], x, W1, degp2)

    accp1 = _sc_prop16(src_pad, dst_pad, u1, np_pad, epw)

    def tc2(accp_ref, u1_ref, dinv_ref, b1_ref, w2_ref, u2_ref):
        acc = (accp_ref[0:np_pad, 0:h1] + accp_ref[np_pad:2 * np_pad, 0:h1]
               - u1_ref[:, 0:h1])
        hh = jnp.maximum(dinv_ref[0:n, :] * acc[0:n, :] + b1_ref[...], 0.0)
        u2_ref[0:n, 0:h2] = jnp.dot(hh, w2_ref[...],
                                    preferred_element_type=jnp.float32) * dinv_ref[0:n, :]
        u2_ref[n:np_pad, 0:h2] = jnp.zeros((np_pad - n, h2), jnp.float32)
        u2_ref[:, h2:16] = jnp.zeros((np_pad, 16 - h2), jnp.float32)

    u2, = _tc_call(tc2, [(np_pad, 16)], accp1, u1, dinv, b1.reshape(1, h1), W2)

    accp2 = _sc_prop16(src_pad, dst_pad, u2, np_pad, epw)

    def tc3(accp_ref, u2_ref, dinv_ref, b2_ref, wc_ref, bc_ref, out_ref, h_ref):
        acc = (accp_ref[0:np_pad, 0:h2] + accp_ref[np_pad:2 * np_pad, 0:h2]
               - u2_ref[:, 0:h2])
        hh = jnp.tanh(dinv_ref[0:n, :] * acc[0:n, :] + b2_ref[...])
        h_ref[...] = hh
        z = jnp.dot(hh, wc_ref[...], preferred_element_type=jnp.float32) + bc_ref[...]
        out_ref[...] = jax.nn.sigmoid(z)

    out, h = _tc_call(tc3, [(n, 1), (n, h2)], accp2, u2, dinv,
                      b2.reshape(1, h2), Wc, bc.reshape(1, 1))
    return (out, h)


# speedup vs baseline: 27.9559x; 1.2702x over previous
import functools

import jax
import jax.numpy as jnp
from jax import lax
from jax.experimental import pallas as pl
from jax.experimental.pallas import tpu as pltpu
from jax.experimental.pallas import tpu_sc as plsc

NC = 2
NS = 16
NW = NC * NS
L = 16
C = 128


def _sc_deg(dst_pad, np_pad, epw):
    nps = np_pad // NS
    nchunk = epw // C
    mesh = plsc.VectorSubcoreMesh(core_axis_name="c", subcore_axis_name="s", num_cores=NC, num_subcores=NS)

    @functools.partial(
        pl.kernel,
        out_type=jax.ShapeDtypeStruct((NC * np_pad,), jnp.float32),
        mesh=mesh,
        scratch_types=[
            pltpu.VMEM((C,), jnp.int32),
            pltpu.VMEM((C,), jnp.float32),
            pltpu.VMEM((nps,), jnp.float32),
            pltpu.VMEM_SHARED((np_pad,), jnp.float32),
        ],
    )
    def k(dst_hbm, out_hbm, didx, ones, zbuf, acc):
        c = lax.axis_index("c")
        s = lax.axis_index("s")
        wid = s * NC + c
        base = wid * epw

        one = jnp.full((L,), 1.0, jnp.float32)
        zero = jnp.zeros((L,), jnp.float32)
        for i in range(C // L):
            ones[pl.ds(i * L, L)] = one
        for i in range(nps // L):
            zbuf[pl.ds(i * L, L)] = zero
        pltpu.sync_copy(zbuf, acc.at[pl.ds(s * nps, nps)])
        plsc.subcore_barrier()

        def body(i, carry):
            pltpu.sync_copy(dst_hbm.at[pl.ds(base + i * C, C)], didx)
            pltpu.sync_copy(ones, acc.at[didx], add=True)
            return carry

        lax.fori_loop(0, nchunk, body, 0)
        plsc.subcore_barrier()
        pltpu.sync_copy(acc.at[pl.ds(s * nps, nps)], zbuf)
        pltpu.sync_copy(zbuf, out_hbm.at[pl.ds(c * np_pad + s * nps, nps)])

    return k(dst_pad)


def _sc_prop(srcf, dstf, u_flat, np_pad, epw, f):
    nf = (np_pad // NS) * f
    nchunk = epw // C
    mesh = plsc.VectorSubcoreMesh(core_axis_name="c", subcore_axis_name="s", num_cores=NC, num_subcores=NS)

    @functools.partial(
        pl.kernel,
        out_type=jax.ShapeDtypeStruct((NC * np_pad * f,), jnp.float32),
        mesh=mesh,
        scratch_types=[
            pltpu.VMEM((C,), jnp.int32),
            pltpu.VMEM((C,), jnp.int32),
            pltpu.VMEM((C,), jnp.float32),
            pltpu.VMEM((nf,), jnp.float32),
            pltpu.VMEM_SHARED((np_pad * f,), jnp.float32),
            pltpu.SemaphoreType.DMA,
        ],
    )
    def k(src_hbm, dst_hbm, u_hbm, out_hbm, sidx, didx, vals, zbuf, acc, sem):
        c = lax.axis_index("c")
        s = lax.axis_index("s")
        wid = s * NC + c
        base = wid * epw

        pltpu.sync_copy(u_hbm.at[pl.ds(s * nf, nf)], zbuf)
        pltpu.sync_copy(zbuf, acc.at[pl.ds(s * nf, nf)])
        plsc.subcore_barrier()

        one = jnp.full((L,), 1, jnp.int32)

        def body(i, carry):
            pltpu.sync_copy(src_hbm.at[pl.ds(base + i * C, C)], sidx)
            pltpu.sync_copy(dst_hbm.at[pl.ds(base + i * C, C)], didx)
            for j in range(f):
                if j > 0:
                    for t in range(C // L):
                        sidx[pl.ds(t * L, L)] = sidx[pl.ds(t * L, L)] + one
                        didx[pl.ds(t * L, L)] = didx[pl.ds(t * L, L)] + one
                pltpu.async_copy(u_hbm.at[sidx], vals, sem).wait()
                pltpu.sync_copy(vals, acc.at[didx], add=True)
            return carry

        lax.fori_loop(0, nchunk, body, 0)
        plsc.subcore_barrier()
        pltpu.sync_copy(acc.at[pl.ds(s * nf, nf)], zbuf)
        pltpu.sync_copy(zbuf, out_hbm.at[pl.ds(c * np_pad * f + s * nf, nf)])

    return k(srcf, dstf, u_flat)


def _sc_prop16(src_pad, dst_pad, u16, np_pad, epw):
    nps = np_pad // NS
    nchunk = epw // C
    mesh = plsc.VectorSubcoreMesh(core_axis_name="c", subcore_axis_name="s", num_cores=NC, num_subcores=NS)

    e_pad = epw * NW

    @functools.partial(
        pl.kernel,
        out_type=jax.ShapeDtypeStruct((NC * np_pad, 16), jnp.float32),
        mesh=mesh,
        scratch_types=[
            pltpu.VMEM((C,), jnp.int32),
            pltpu.VMEM((C,), jnp.int32),
            pltpu.VMEM((C,), jnp.int32),
            pltpu.VMEM((C,), jnp.int32),
            pltpu.VMEM((C, 16), jnp.float32),
            pltpu.VMEM((C, 16), jnp.float32),
            pltpu.VMEM((nps, 16), jnp.float32),
            pltpu.VMEM_SHARED((np_pad, 16), jnp.float32),
            pltpu.VMEM_SHARED((np_pad, 16), jnp.float32),
            pltpu.SemaphoreType.DMA,
            pltpu.SemaphoreType.DMA,
            pltpu.SemaphoreType.DMA,
            pltpu.SemaphoreType.DMA,
        ],
        compiler_params=pltpu.CompilerParams(use_tc_tiling_on_sc=False),
    )
    def k(src_hbm, dst_hbm, u_hbm, out_hbm,
          sidx0, sidx1, didx0, didx1, rows0, rows1, zbuf, acc, uloc,
          gsem0, gsem1, ssem0, ssem1):
        c = lax.axis_index("c")
        s = lax.axis_index("s")
        wid = s * NC + c
        base = wid * epw
        bufs = ((sidx0, didx0, rows0, gsem0, ssem0),
                (sidx1, didx1, rows1, gsem1, ssem1))

        pltpu.sync_copy(u_hbm.at[pl.ds(s * nps, nps)], zbuf)
        pltpu.sync_copy(zbuf, acc.at[pl.ds(s * nps, nps)])
        pltpu.sync_copy(zbuf, uloc.at[pl.ds(s * nps, nps)])
        plsc.subcore_barrier()

        (sidx0, didx0, rows0, gsem0, _s0), (sidx1, didx1, rows1, gsem1, _s1) = bufs

        def body(i, carry):
            off0 = base + 2 * i * C
            pltpu.sync_copy(src_hbm.at[pl.ds(off0, C)], sidx0)
            pltpu.sync_copy(dst_hbm.at[pl.ds(off0, C)], didx0)
            g0 = pltpu.async_copy(uloc.at[sidx0], rows0, gsem0)
            pltpu.sync_copy(src_hbm.at[pl.ds(off0 + C, C)], sidx1)
            pltpu.sync_copy(dst_hbm.at[pl.ds(off0 + C, C)], didx1)
            g0.wait()
            g1 = pltpu.async_copy(uloc.at[sidx1], rows1, gsem1)
            g1.wait()
            pltpu.sync_copy(rows0, acc.at[didx0], add=True)
            pltpu.sync_copy(rows1, acc.at[didx1], add=True)
            return carry

        lax.fori_loop(0, nchunk // 2, body, 0)
        plsc.subcore_barrier()
        pltpu.sync_copy(acc.at[pl.ds(s * nps, nps)], zbuf)
        pltpu.sync_copy(zbuf, out_hbm.at[pl.ds(c * np_pad + s * nps, nps)])

    return k(src_pad, dst_pad, u16)


def _tc_call(body, out_shapes, *args):
    return pl.pallas_call(
        body,
        out_shape=[jax.ShapeDtypeStruct(s, jnp.float32) for s in out_shapes],
    )(*args)


def kernel(x, edge_index, W1, b1, W2, b2, Wc, bc):
    n, d = x.shape
    e = edge_index.shape[1]
    h1 = W1.shape[1]
    h2 = W2.shape[1]

    align = NS * L
    np_pad = ((n + align - 1) // (align)) * align
    while np_pad - n < C:
        np_pad += align
    e_pad = ((e + 2 * NW * C - 1) // (2 * NW * C)) * (2 * NW * C)
    epw = e_pad // NW

    src = edge_index[0]
    dst = edge_index[1]
    pad = e_pad - e
    src_pad = jnp.concatenate([src, jnp.zeros((pad,), jnp.int32)])
    dst_pad = jnp.concatenate([dst, jnp.full((pad,), np_pad - 1, jnp.int32)])

    degp = _sc_deg(dst_pad, np_pad, epw)
    degp2 = degp.reshape((NC * np_pad, 1))

    def tc1(x_ref, w1_ref, degp_ref, u1_ref, dinv_ref):
        deg = degp_ref[0:np_pad, :] + degp_ref[np_pad:2 * np_pad, :] + 1.0
        dinv = lax.rsqrt(deg)
        dinv_ref[...] = dinv
        hw = jnp.dot(x_ref[...], w1_ref[...], preferred_element_type=jnp.float32)
        u1_ref[0:n, 0:h1] = hw * dinv[0:n, :]
        u1_ref[n:np_pad, 0:h1] = jnp.zeros((np_pad - n, h1), jnp.float32)
        u1_ref[:, h1:16] = jnp.zeros((np_pad, 16 - h1), jnp.float32)

    u1, dinv = _tc_call(tc1, [(np_pad, 16), (np_pad, 1)], x, W1, degp2)

    accp1 = _sc_prop16(src_pad, dst_pad, u1, np_pad, epw)

    def tc2(accp_ref, u1_ref, dinv_ref, b1_ref, w2_ref, u2_ref):
        acc = (accp_ref[0:np_pad, 0:h1] + accp_ref[np_pad:2 * np_pad, 0:h1]
               - u1_ref[:, 0:h1])
        hh = jnp.maximum(dinv_ref[0:n, :] * acc[0:n, :] + b1_ref[...], 0.0)
        u2_ref[0:n, 0:h2] = jnp.dot(hh, w2_ref[...],
                                    preferred_element_type=jnp.float32) * dinv_ref[0:n, :]
        u2_ref[n:np_pad, 0:h2] = jnp.zeros((np_pad - n, h2), jnp.float32)
        u2_ref[:, h2:16] = jnp.zeros((np_pad, 16 - h2), jnp.float32)

    u2, = _tc_call(tc2, [(np_pad, 16)], accp1, u1, dinv, b1.reshape(1, h1), W2)

    accp2 = _sc_prop16(src_pad, dst_pad, u2, np_pad, epw)

    def tc3(accp_ref, u2_ref, dinv_ref, b2_ref, wc_ref, bc_ref, out_ref, h_ref):
        acc = (accp_ref[0:np_pad, 0:h2] + accp_ref[np_pad:2 * np_pad, 0:h2]
               - u2_ref[:, 0:h2])
        hh = jnp.tanh(dinv_ref[0:n, :] * acc[0:n, :] + b2_ref[...])
        h_ref[...] = hh
        z = jnp.dot(hh, wc_ref[...], preferred_element_type=jnp.float32) + bc_ref[...]
        out_ref[...] = jax.nn.sigmoid(z)

    out, h = _tc_call(tc3, [(n, 1), (n, h2)], accp2, u2, dinv,
                      b2.reshape(1, h2), Wc, bc.reshape(1, 1))
    return (out, h)


# speedup vs baseline: 53.9305x; 1.9291x over previous
import functools

import jax
import jax.numpy as jnp
from jax import lax
from jax.experimental import pallas as pl
from jax.experimental.pallas import tpu as pltpu
from jax.experimental.pallas import tpu_sc as plsc

NC = 2
NS = 16
NW = NC * NS
L = 16
C = 128


def _sc_deg(dst3, np_pad, epw):
    nps = np_pad // NS
    nchunk = epw // C
    mesh = plsc.VectorSubcoreMesh(core_axis_name="c", subcore_axis_name="s", num_cores=NC, num_subcores=NS)

    @functools.partial(
        pl.kernel,
        out_type=jax.ShapeDtypeStruct((NC * np_pad,), jnp.float32),
        mesh=mesh,
        scratch_types=[
            pltpu.VMEM((nchunk, C), jnp.int32),
            pltpu.VMEM((C,), jnp.float32),
            pltpu.VMEM((nps,), jnp.float32),
            pltpu.VMEM_SHARED((np_pad,), jnp.float32),
            pltpu.SemaphoreType.DMA,
        ],
    )
    def k(dst_hbm, out_hbm, dbuf, ones, zbuf, acc, lsem):
        c = lax.axis_index("c")
        s = lax.axis_index("s")
        wid = s * NC + c

        ld = pltpu.async_copy(dst_hbm.at[wid], dbuf, lsem)
        one = jnp.full((L,), 1.0, jnp.float32)
        zero = jnp.zeros((L,), jnp.float32)
        for i in range(C // L):
            ones[pl.ds(i * L, L)] = one
        for i in range(nps // L):
            zbuf[pl.ds(i * L, L)] = zero
        pltpu.sync_copy(zbuf, acc.at[pl.ds(s * nps, nps)])
        ld.wait()
        plsc.subcore_barrier()

        def body(i, carry):
            pltpu.sync_copy(ones, acc.at[dbuf.at[i]], add=True)
            return carry

        lax.fori_loop(0, nchunk, body, 0)
        plsc.subcore_barrier()
        pltpu.sync_copy(acc.at[pl.ds(s * nps, nps)], zbuf)
        pltpu.sync_copy(zbuf, out_hbm.at[pl.ds(c * np_pad + s * nps, nps)])

    return k(dst3)


def _sc_prop(srcf, dstf, u_flat, np_pad, epw, f):
    nf = (np_pad // NS) * f
    nchunk = epw // C
    mesh = plsc.VectorSubcoreMesh(core_axis_name="c", subcore_axis_name="s", num_cores=NC, num_subcores=NS)

    @functools.partial(
        pl.kernel,
        out_type=jax.ShapeDtypeStruct((NC * np_pad * f,), jnp.float32),
        mesh=mesh,
        scratch_types=[
            pltpu.VMEM((C,), jnp.int32),
            pltpu.VMEM((C,), jnp.int32),
            pltpu.VMEM((C,), jnp.float32),
            pltpu.VMEM((nf,), jnp.float32),
            pltpu.VMEM_SHARED((np_pad * f,), jnp.float32),
            pltpu.SemaphoreType.DMA,
        ],
    )
    def k(src_hbm, dst_hbm, u_hbm, out_hbm, sidx, didx, vals, zbuf, acc, sem):
        c = lax.axis_index("c")
        s = lax.axis_index("s")
        wid = s * NC + c
        base = wid * epw

        pltpu.sync_copy(u_hbm.at[pl.ds(s * nf, nf)], zbuf)
        pltpu.sync_copy(zbuf, acc.at[pl.ds(s * nf, nf)])
        plsc.subcore_barrier()

        one = jnp.full((L,), 1, jnp.int32)

        def body(i, carry):
            pltpu.sync_copy(src_hbm.at[pl.ds(base + i * C, C)], sidx)
            pltpu.sync_copy(dst_hbm.at[pl.ds(base + i * C, C)], didx)
            for j in range(f):
                if j > 0:
                    for t in range(C // L):
                        sidx[pl.ds(t * L, L)] = sidx[pl.ds(t * L, L)] + one
                        didx[pl.ds(t * L, L)] = didx[pl.ds(t * L, L)] + one
                pltpu.async_copy(u_hbm.at[sidx], vals, sem).wait()
                pltpu.sync_copy(vals, acc.at[didx], add=True)
            return carry

        lax.fori_loop(0, nchunk, body, 0)
        plsc.subcore_barrier()
        pltpu.sync_copy(acc.at[pl.ds(s * nf, nf)], zbuf)
        pltpu.sync_copy(zbuf, out_hbm.at[pl.ds(c * np_pad * f + s * nf, nf)])

    return k(srcf, dstf, u_flat)


def _sc_prop16(src3, dst3, u16, np_pad, epw):
    nps = np_pad // NS
    nchunk = epw // C
    mesh = plsc.VectorSubcoreMesh(core_axis_name="c", subcore_axis_name="s", num_cores=NC, num_subcores=NS)

    @functools.partial(
        pl.kernel,
        out_type=jax.ShapeDtypeStruct((NC * np_pad, 16), jnp.float32),
        mesh=mesh,
        scratch_types=[
            pltpu.VMEM((nchunk, C), jnp.int32),
            pltpu.VMEM((nchunk, C), jnp.int32),
            pltpu.VMEM((C, 16), jnp.float32),
            pltpu.VMEM((C, 16), jnp.float32),
            pltpu.VMEM((nps, 16), jnp.float32),
            pltpu.VMEM_SHARED((np_pad, 16), jnp.float32),
            pltpu.VMEM_SHARED((np_pad, 16), jnp.float32),
            pltpu.SemaphoreType.DMA,
            pltpu.SemaphoreType.DMA,
            pltpu.SemaphoreType.DMA,
            pltpu.SemaphoreType.DMA,
        ],
        compiler_params=pltpu.CompilerParams(use_tc_tiling_on_sc=False),
    )
    def k(src_hbm, dst_hbm, u_hbm, out_hbm,
          sbuf, dbuf, rows0, rows1, zbuf, acc, uloc,
          lsem0, lsem1, gsem0, gsem1):
        c = lax.axis_index("c")
        s = lax.axis_index("s")
        wid = s * NC + c

        ls = pltpu.async_copy(src_hbm.at[wid], sbuf, lsem0)
        ld = pltpu.async_copy(dst_hbm.at[wid], dbuf, lsem1)
        pltpu.sync_copy(u_hbm.at[pl.ds(s * nps, nps)], zbuf)
        pltpu.sync_copy(zbuf, acc.at[pl.ds(s * nps, nps)])
        pltpu.sync_copy(zbuf, uloc.at[pl.ds(s * nps, nps)])
        ls.wait()
        ld.wait()
        plsc.subcore_barrier()

        def body(i, carry):
            g0 = pltpu.async_copy(uloc.at[sbuf.at[2 * i]], rows0, gsem0)
            g1 = pltpu.async_copy(uloc.at[sbuf.at[2 * i + 1]], rows1, gsem1)
            g0.wait()
            pltpu.sync_copy(rows0, acc.at[dbuf.at[2 * i]], add=True)
            g1.wait()
            pltpu.sync_copy(rows1, acc.at[dbuf.at[2 * i + 1]], add=True)
            return carry

        lax.fori_loop(0, nchunk // 2, body, 0)
        plsc.subcore_barrier()
        pltpu.sync_copy(acc.at[pl.ds(s * nps, nps)], zbuf)
        pltpu.sync_copy(zbuf, out_hbm.at[pl.ds(c * np_pad + s * nps, nps)])

    return k(src3, dst3, u16)


def _tc_call(body, out_shapes, *args):
    return pl.pallas_call(
        body,
        out_shape=[jax.ShapeDtypeStruct(s, jnp.float32) for s in out_shapes],
    )(*args)


def kernel(x, edge_index, W1, b1, W2, b2, Wc, bc):
    n, d = x.shape
    e = edge_index.shape[1]
    h1 = W1.shape[1]
    h2 = W2.shape[1]

    align = NS * L
    np_pad = ((n + align - 1) // (align)) * align
    while np_pad - n < C:
        np_pad += align
    e_pad = ((e + 2 * NW * C - 1) // (2 * NW * C)) * (2 * NW * C)
    epw = e_pad // NW

    src = edge_index[0]
    dst = edge_index[1]
    pad = e_pad - e
    nchunk = epw // C
    src3 = jnp.concatenate([src, jnp.zeros((pad,), jnp.int32)]).reshape(NW, nchunk, C)
    dst3 = jnp.concatenate([dst, jnp.full((pad,), np_pad - 1, jnp.int32)]).reshape(NW, nchunk, C)

    degp = _sc_deg(dst3, np_pad, epw)
    degp2 = degp.reshape((NC * np_pad, 1))

    def tc1(x_ref, w1_ref, degp_ref, u1_ref, dinv_ref):
        deg = degp_ref[0:np_pad, :] + degp_ref[np_pad:2 * np_pad, :] + 1.0
        dinv = lax.rsqrt(deg)
        dinv_ref[...] = dinv
        hw = jnp.dot(x_ref[...], w1_ref[...], preferred_element_type=jnp.float32)
        u1_ref[0:n, 0:h1] = hw * dinv[0:n, :]
        u1_ref[n:np_pad, 0:h1] = jnp.zeros((np_pad - n, h1), jnp.float32)
        u1_ref[:, h1:16] = jnp.zeros((np_pad, 16 - h1), jnp.float32)

    u1, dinv = _tc_call(tc1, [(np_pad, 16), (np_pad, 1)], x, W1, degp2)

    accp1 = _sc_prop16(src3, dst3, u1, np_pad, epw)

    def tc2(accp_ref, u1_ref, dinv_ref, b1_ref, w2_ref, u2_ref):
        acc = (accp_ref[0:np_pad, 0:h1] + accp_ref[np_pad:2 * np_pad, 0:h1]
               - u1_ref[:, 0:h1])
        hh = jnp.maximum(dinv_ref[0:n, :] * acc[0:n, :] + b1_ref[...], 0.0)
        u2_ref[0:n, 0:h2] = jnp.dot(hh, w2_ref[...],
                                    preferred_element_type=jnp.float32) * dinv_ref[0:n, :]
        u2_ref[n:np_pad, 0:h2] = jnp.zeros((np_pad - n, h2), jnp.float32)
        u2_ref[:, h2:16] = jnp.zeros((np_pad, 16 - h2), jnp.float32)

    u2, = _tc_call(tc2, [(np_pad, 16)], accp1, u1, dinv, b1.reshape(1, h1), W2)

    accp2 = _sc_prop16(src3, dst3, u2, np_pad, epw)

    def tc3(accp_ref, u2_ref, dinv_ref, b2_ref, wc_ref, bc_ref, out_ref, h_ref):
        acc = (accp_ref[0:np_pad, 0:h2] + accp_ref[np_pad:2 * np_pad, 0:h2]
               - u2_ref[:, 0:h2])
        hh = jnp.tanh(dinv_ref[0:n, :] * acc[0:n, :] + b2_ref[...])
        h_ref[...] = hh
        z = jnp.dot(hh, wc_ref[...], preferred_element_type=jnp.float32) + bc_ref[...]
        out_ref[...] = jax.nn.sigmoid(z)

    out, h = _tc_call(tc3, [(n, 1), (n, h2)], accp2, u2, dinv,
                      b2.reshape(1, h2), Wc, bc.reshape(1, 1))
    return (out, h)


# speedup vs baseline: 53.9324x; 1.0000x over previous
import functools

import jax
import jax.numpy as jnp
from jax import lax
from jax.experimental import pallas as pl
from jax.experimental.pallas import tpu as pltpu
from jax.experimental.pallas import tpu_sc as plsc

NC = 2
NS = 16
NW = NC * NS
L = 16
C = 128


def _sc_deg(dst3, np_pad, epw):
    nps = np_pad // NS
    nchunk = epw // C
    mesh = plsc.VectorSubcoreMesh(core_axis_name="c", subcore_axis_name="s", num_cores=NC, num_subcores=NS)

    @functools.partial(
        pl.kernel,
        out_type=jax.ShapeDtypeStruct((NC * np_pad,), jnp.float32),
        mesh=mesh,
        scratch_types=[
            pltpu.VMEM((nchunk, C), jnp.int32),
            pltpu.VMEM((C,), jnp.float32),
            pltpu.VMEM((nps,), jnp.float32),
            pltpu.VMEM_SHARED((np_pad,), jnp.float32),
            pltpu.SemaphoreType.DMA,
        ],
    )
    def k(dst_hbm, out_hbm, dbuf, ones, zbuf, acc, lsem):
        c = lax.axis_index("c")
        s = lax.axis_index("s")
        wid = s * NC + c

        ld = pltpu.async_copy(dst_hbm.at[wid], dbuf, lsem)
        one = jnp.full((L,), 1.0, jnp.float32)
        zero = jnp.zeros((L,), jnp.float32)
        for i in range(C // L):
            ones[pl.ds(i * L, L)] = one
        for i in range(nps // L):
            zbuf[pl.ds(i * L, L)] = zero
        pltpu.sync_copy(zbuf, acc.at[pl.ds(s * nps, nps)])
        ld.wait()
        plsc.subcore_barrier()

        def body(i, carry):
            pltpu.sync_copy(ones, acc.at[dbuf.at[i]], add=True)
            return carry

        lax.fori_loop(0, nchunk, body, 0)
        plsc.subcore_barrier()
        pltpu.sync_copy(acc.at[pl.ds(s * nps, nps)], zbuf)
        pltpu.sync_copy(zbuf, out_hbm.at[pl.ds(c * np_pad + s * nps, nps)])

    return k(dst3)


def _sc_prop(srcf, dstf, u_flat, np_pad, epw, f):
    nf = (np_pad // NS) * f
    nchunk = epw // C
    mesh = plsc.VectorSubcoreMesh(core_axis_name="c", subcore_axis_name="s", num_cores=NC, num_subcores=NS)

    @functools.partial(
        pl.kernel,
        out_type=jax.ShapeDtypeStruct((NC * np_pad * f,), jnp.float32),
        mesh=mesh,
        scratch_types=[
            pltpu.VMEM((C,), jnp.int32),
            pltpu.VMEM((C,), jnp.int32),
            pltpu.VMEM((C,), jnp.float32),
            pltpu.VMEM((nf,), jnp.float32),
            pltpu.VMEM_SHARED((np_pad * f,), jnp.float32),
            pltpu.SemaphoreType.DMA,
        ],
    )
    def k(src_hbm, dst_hbm, u_hbm, out_hbm, sidx, didx, vals, zbuf, acc, sem):
        c = lax.axis_index("c")
        s = lax.axis_index("s")
        wid = s * NC + c
        base = wid * epw

        pltpu.sync_copy(u_hbm.at[pl.ds(s * nf, nf)], zbuf)
        pltpu.sync_copy(zbuf, acc.at[pl.ds(s * nf, nf)])
        plsc.subcore_barrier()

        one = jnp.full((L,), 1, jnp.int32)

        def body(i, carry):
            pltpu.sync_copy(src_hbm.at[pl.ds(base + i * C, C)], sidx)
            pltpu.sync_copy(dst_hbm.at[pl.ds(base + i * C, C)], didx)
            for j in range(f):
                if j > 0:
                    for t in range(C // L):
                        sidx[pl.ds(t * L, L)] = sidx[pl.ds(t * L, L)] + one
                        didx[pl.ds(t * L, L)] = didx[pl.ds(t * L, L)] + one
                pltpu.async_copy(u_hbm.at[sidx], vals, sem).wait()
                pltpu.sync_copy(vals, acc.at[didx], add=True)
            return carry

        lax.fori_loop(0, nchunk, body, 0)
        plsc.subcore_barrier()
        pltpu.sync_copy(acc.at[pl.ds(s * nf, nf)], zbuf)
        pltpu.sync_copy(zbuf, out_hbm.at[pl.ds(c * np_pad * f + s * nf, nf)])

    return k(srcf, dstf, u_flat)


def _sc_prop16(src3, dst3, u16, np_pad, epw):
    nps = np_pad // NS
    nchunk = epw // C
    mesh = plsc.VectorSubcoreMesh(core_axis_name="c", subcore_axis_name="s", num_cores=NC, num_subcores=NS)

    @functools.partial(
        pl.kernel,
        out_type=jax.ShapeDtypeStruct((NC * np_pad, 16), jnp.float32),
        mesh=mesh,
        scratch_types=[
            pltpu.VMEM((nchunk, C), jnp.int32),
            pltpu.VMEM((nchunk, C), jnp.int32),
            pltpu.VMEM((C, 16), jnp.float32),
            pltpu.VMEM((C, 16), jnp.float32),
            pltpu.VMEM((nps, 16), jnp.float32),
            pltpu.VMEM_SHARED((np_pad, 16), jnp.float32),
            pltpu.VMEM_SHARED((np_pad, 16), jnp.float32),
            pltpu.SemaphoreType.DMA,
            pltpu.SemaphoreType.DMA,
            pltpu.SemaphoreType.DMA,
            pltpu.SemaphoreType.DMA,
        ],
        compiler_params=pltpu.CompilerParams(use_tc_tiling_on_sc=False),
    )
    def k(src_hbm, dst_hbm, u_hbm, out_hbm,
          sbuf, dbuf, rows0, rows1, zbuf, acc, uloc,
          lsem0, lsem1, gsem0, gsem1):
        c = lax.axis_index("c")
        s = lax.axis_index("s")
        wid = s * NC + c

        ls = pltpu.async_copy(src_hbm.at[wid], sbuf, lsem0)
        ld = pltpu.async_copy(dst_hbm.at[wid], dbuf, lsem1)
        pltpu.sync_copy(u_hbm.at[pl.ds(s * nps, nps)], zbuf)
        pltpu.sync_copy(zbuf, acc.at[pl.ds(s * nps, nps)])
        pltpu.sync_copy(zbuf, uloc.at[pl.ds(s * nps, nps)])
        ls.wait()
        ld.wait()
        plsc.subcore_barrier()

        def body(i, carry):
            g0 = pltpu.async_copy(uloc.at[sbuf.at[2 * i]], rows0, gsem0)
            g1 = pltpu.async_copy(uloc.at[sbuf.at[2 * i + 1]], rows1, gsem1)
            g0.wait()
            pltpu.sync_copy(rows0, acc.at[dbuf.at[2 * i]], add=True)
            g1.wait()
            pltpu.sync_copy(rows1, acc.at[dbuf.at[2 * i + 1]], add=True)
            return carry

        lax.fori_loop(0, nchunk // 2, body, 0)
        plsc.subcore_barrier()
        pltpu.sync_copy(acc.at[pl.ds(s * nps, nps)], zbuf)
        pltpu.sync_copy(zbuf, out_hbm.at[pl.ds(c * np_pad + s * nps, nps)])

    return k(src3, dst3, u16)


def _tc_call(body, out_shapes, *args):
    return pl.pallas_call(
        body,
        out_shape=[jax.ShapeDtypeStruct(s, jnp.float32) for s in out_shapes],
    )(*args)


def kernel(x, edge_index, W1, b1, W2, b2, Wc, bc):
    n, d = x.shape
    e = edge_index.shape[1]
    h1 = W1.shape[1]
    h2 = W2.shape[1]

    align = NS * L
    np_pad = ((n + align - 1) // (align)) * align
    while np_pad - n < C:
        np_pad += align
    e_pad = ((e + 2 * NW * C - 1) // (2 * NW * C)) * (2 * NW * C)
    epw = e_pad // NW

    src = edge_index[0]
    dst = edge_index[1]
    pad = e_pad - e
    nchunk = epw // C
    src3 = jnp.concatenate([src, jnp.zeros((pad,), jnp.int32)]).reshape(NW, nchunk, C)
    dst3 = jnp.concatenate([dst, jnp.full((pad,), np_pad - 1, jnp.int32)]).reshape(NW, nchunk, C)

    degp = _sc_deg(dst3, np_pad, epw)
    degp2 = degp.reshape((NC * np_pad, 1))

    def tc1a(x_ref, w1_ref, hw_ref):
        hw_ref[...] = jnp.dot(x_ref[...], w1_ref[...],
                              preferred_element_type=jnp.float32)

    hw, = _tc_call(tc1a, [(n, h1)], x, W1)

    def tc1b(hw_ref, degp_ref, u1_ref, dinv_ref):
        deg = degp_ref[0:np_pad, :] + degp_ref[np_pad:2 * np_pad, :] + 1.0
        dinv = lax.rsqrt(deg)
        dinv_ref[...] = dinv
        u1_ref[0:n, 0:h1] = hw_ref[...] * dinv[0:n, :]
        u1_ref[n:np_pad, 0:h1] = jnp.zeros((np_pad - n, h1), jnp.float32)
        u1_ref[:, h1:16] = jnp.zeros((np_pad, 16 - h1), jnp.float32)

    u1, dinv = _tc_call(tc1b, [(np_pad, 16), (np_pad, 1)], hw, degp2)

    accp1 = _sc_prop16(src3, dst3, u1, np_pad, epw)

    def tc2(accp_ref, u1_ref, dinv_ref, b1_ref, w2_ref, u2_ref):
        acc = (accp_ref[0:np_pad, 0:h1] + accp_ref[np_pad:2 * np_pad, 0:h1]
               - u1_ref[:, 0:h1])
        hh = jnp.maximum(dinv_ref[0:n, :] * acc[0:n, :] + b1_ref[...], 0.0)
        u2_ref[0:n, 0:h2] = jnp.dot(hh, w2_ref[...],
                                    preferred_element_type=jnp.float32) * dinv_ref[0:n, :]
        u2_ref[n:np_pad, 0:h2] = jnp.zeros((np_pad - n, h2), jnp.float32)
        u2_ref[:, h2:16] = jnp.zeros((np_pad, 16 - h2), jnp.float32)

    u2, = _tc_call(tc2, [(np_pad, 16)], accp1, u1, dinv, b1.reshape(1, h1), W2)

    accp2 = _sc_prop16(src3, dst3, u2, np_pad, epw)

    def tc3(accp_ref, u2_ref, dinv_ref, b2_ref, wc_ref, bc_ref, out_ref, h_ref):
        acc = (accp_ref[0:np_pad, 0:h2] + accp_ref[np_pad:2 * np_pad, 0:h2]
               - u2_ref[:, 0:h2])
        hh = jnp.tanh(dinv_ref[0:n, :] * acc[0:n, :] + b2_ref[...])
        h_ref[...] = hh
        z = jnp.dot(hh, wc_ref[...], preferred_element_type=jnp.float32) + bc_ref[...]
        out_ref[...] = jax.nn.sigmoid(z)

    out, h = _tc_call(tc3, [(n, 1), (n, h2)], accp2, u2, dinv,
                      b2.reshape(1, h2), Wc, bc.reshape(1, 1))
    return (out, h)


# speedup vs baseline: 56.9054x; 1.0551x over previous
import functools

import jax
import jax.numpy as jnp
from jax import lax
from jax.experimental import pallas as pl
from jax.experimental.pallas import tpu as pltpu
from jax.experimental.pallas import tpu_sc as plsc

NC = 2
NS = 16
NW = NC * NS
L = 16
C = 128


def _sc_deg(dst3, np_pad, epw):
    nps = np_pad // NS
    nchunk = epw // C
    mesh = plsc.VectorSubcoreMesh(core_axis_name="c", subcore_axis_name="s", num_cores=NC, num_subcores=NS)

    @functools.partial(
        pl.kernel,
        out_type=jax.ShapeDtypeStruct((NC * np_pad,), jnp.float32),
        mesh=mesh,
        scratch_types=[
            pltpu.VMEM((nchunk, C), jnp.int32),
            pltpu.VMEM((C,), jnp.float32),
            pltpu.VMEM((nps,), jnp.float32),
            pltpu.VMEM_SHARED((np_pad,), jnp.float32),
            pltpu.SemaphoreType.DMA,
        ],
    )
    def k(dst_hbm, out_hbm, dbuf, ones, zbuf, acc, lsem):
        c = lax.axis_index("c")
        s = lax.axis_index("s")
        wid = s * NC + c

        ld = pltpu.async_copy(dst_hbm.at[wid], dbuf, lsem)
        one = jnp.full((L,), 1.0, jnp.float32)
        zero = jnp.zeros((L,), jnp.float32)
        for i in range(C // L):
            ones[pl.ds(i * L, L)] = one
        for i in range(nps // L):
            zbuf[pl.ds(i * L, L)] = zero
        pltpu.sync_copy(zbuf, acc.at[pl.ds(s * nps, nps)])
        ld.wait()
        plsc.subcore_barrier()

        def body(i, carry):
            pltpu.sync_copy(ones, acc.at[dbuf.at[i]], add=True)
            return carry

        lax.fori_loop(0, nchunk, body, 0)
        plsc.subcore_barrier()
        pltpu.sync_copy(acc.at[pl.ds(s * nps, nps)], zbuf)
        pltpu.sync_copy(zbuf, out_hbm.at[pl.ds(c * np_pad + s * nps, nps)])

    return k(dst3)


def _sc_prop(srcf, dstf, u_flat, np_pad, epw, f):
    nf = (np_pad // NS) * f
    nchunk = epw // C
    mesh = plsc.VectorSubcoreMesh(core_axis_name="c", subcore_axis_name="s", num_cores=NC, num_subcores=NS)

    @functools.partial(
        pl.kernel,
        out_type=jax.ShapeDtypeStruct((NC * np_pad * f,), jnp.float32),
        mesh=mesh,
        scratch_types=[
            pltpu.VMEM((C,), jnp.int32),
            pltpu.VMEM((C,), jnp.int32),
            pltpu.VMEM((C,), jnp.float32),
            pltpu.VMEM((nf,), jnp.float32),
            pltpu.VMEM_SHARED((np_pad * f,), jnp.float32),
            pltpu.SemaphoreType.DMA,
        ],
    )
    def k(src_hbm, dst_hbm, u_hbm, out_hbm, sidx, didx, vals, zbuf, acc, sem):
        c = lax.axis_index("c")
        s = lax.axis_index("s")
        wid = s * NC + c
        base = wid * epw

        pltpu.sync_copy(u_hbm.at[pl.ds(s * nf, nf)], zbuf)
        pltpu.sync_copy(zbuf, acc.at[pl.ds(s * nf, nf)])
        plsc.subcore_barrier()

        one = jnp.full((L,), 1, jnp.int32)

        def body(i, carry):
            pltpu.sync_copy(src_hbm.at[pl.ds(base + i * C, C)], sidx)
            pltpu.sync_copy(dst_hbm.at[pl.ds(base + i * C, C)], didx)
            for j in range(f):
                if j > 0:
                    for t in range(C // L):
                        sidx[pl.ds(t * L, L)] = sidx[pl.ds(t * L, L)] + one
                        didx[pl.ds(t * L, L)] = didx[pl.ds(t * L, L)] + one
                pltpu.async_copy(u_hbm.at[sidx], vals, sem).wait()
                pltpu.sync_copy(vals, acc.at[didx], add=True)
            return carry

        lax.fori_loop(0, nchunk, body, 0)
        plsc.subcore_barrier()
        pltpu.sync_copy(acc.at[pl.ds(s * nf, nf)], zbuf)
        pltpu.sync_copy(zbuf, out_hbm.at[pl.ds(c * np_pad * f + s * nf, nf)])

    return k(srcf, dstf, u_flat)


def _sc_prop16(src3, dst3, u16, np_pad, epw):
    nps = np_pad // NS
    nchunk = epw // C
    mesh = plsc.VectorSubcoreMesh(core_axis_name="c", subcore_axis_name="s", num_cores=NC, num_subcores=NS)

    @functools.partial(
        pl.kernel,
        out_type=jax.ShapeDtypeStruct((NC * np_pad, 16), jnp.float32),
        mesh=mesh,
        scratch_types=[
            pltpu.VMEM((nchunk, C), jnp.int32),
            pltpu.VMEM((nchunk, C), jnp.int32),
            pltpu.VMEM((C, 16), jnp.float32),
            pltpu.VMEM((C, 16), jnp.float32),
            pltpu.VMEM((C, 16), jnp.float32),
            pltpu.VMEM((C, 16), jnp.float32),
            pltpu.VMEM((nps, 16), jnp.float32),
            pltpu.VMEM_SHARED((np_pad, 16), jnp.float32),
            pltpu.VMEM_SHARED((np_pad, 16), jnp.float32),
            pltpu.SemaphoreType.DMA,
            pltpu.SemaphoreType.DMA,
            pltpu.SemaphoreType.DMA,
            pltpu.SemaphoreType.DMA,
            pltpu.SemaphoreType.DMA,
            pltpu.SemaphoreType.DMA,
        ],
        compiler_params=pltpu.CompilerParams(use_tc_tiling_on_sc=False),
    )
    def k(src_hbm, dst_hbm, u_hbm, out_hbm,
          sbuf, dbuf, rows0, rows1, rows2, rows3, zbuf, acc, uloc,
          lsem0, lsem1, gsem0, gsem1, gsem2, gsem3):
        c = lax.axis_index("c")
        s = lax.axis_index("s")
        wid = s * NC + c

        ls = pltpu.async_copy(src_hbm.at[wid], sbuf, lsem0)
        ld = pltpu.async_copy(dst_hbm.at[wid], dbuf, lsem1)
        pltpu.sync_copy(u_hbm.at[pl.ds(s * nps, nps)], zbuf)
        pltpu.sync_copy(zbuf, acc.at[pl.ds(s * nps, nps)])
        pltpu.sync_copy(zbuf, uloc.at[pl.ds(s * nps, nps)])
        ls.wait()
        ld.wait()
        plsc.subcore_barrier()

        rb = (rows0, rows1, rows2, rows3)
        sm = (gsem0, gsem1, gsem2, gsem3)

        def body(i, carry):
            b = i * 8
            hs = [None] * 8
            for j in range(4):
                hs[j] = pltpu.async_copy(uloc.at[sbuf.at[b + j]], rb[j], sm[j])
            for j in range(8):
                hs[j].wait()
                pltpu.sync_copy(rb[j % 4], acc.at[dbuf.at[b + j]], add=True)
                if j + 4 < 8:
                    hs[j + 4] = pltpu.async_copy(
                        uloc.at[sbuf.at[b + j + 4]], rb[j % 4], sm[j % 4])
            return carry

        lax.fori_loop(0, nchunk // 8, body, 0)
        plsc.subcore_barrier()
        pltpu.sync_copy(acc.at[pl.ds(s * nps, nps)], zbuf)
        pltpu.sync_copy(zbuf, out_hbm.at[pl.ds(c * np_pad + s * nps, nps)])

    return k(src3, dst3, u16)


def _tc_call(body, out_shapes, *args):
    return pl.pallas_call(
        body,
        out_shape=[jax.ShapeDtypeStruct(s, jnp.float32) for s in out_shapes],
    )(*args)


def kernel(x, edge_index, W1, b1, W2, b2, Wc, bc):
    n, d = x.shape
    e = edge_index.shape[1]
    h1 = W1.shape[1]
    h2 = W2.shape[1]

    align = NS * L
    np_pad = ((n + align - 1) // (align)) * align
    while np_pad - n < C:
        np_pad += align
    e_pad = ((e + 8 * NW * C - 1) // (8 * NW * C)) * (8 * NW * C)
    epw = e_pad // NW

    src = edge_index[0]
    dst = edge_index[1]
    pad = e_pad - e
    nchunk = epw // C
    src3 = jnp.concatenate([src, jnp.zeros((pad,), jnp.int32)]).reshape(NW, nchunk, C)
    dst3 = jnp.concatenate([dst, jnp.full((pad,), np_pad - 1, jnp.int32)]).reshape(NW, nchunk, C)

    degp = _sc_deg(dst3, np_pad, epw)
    degp2 = degp.reshape((NC * np_pad, 1))

    def tc1(x_ref, w1_ref, degp_ref, u1_ref, dinv_ref):
        deg = degp_ref[0:np_pad, :] + degp_ref[np_pad:2 * np_pad, :] + 1.0
        dinv = lax.rsqrt(deg)
        dinv_ref[...] = dinv
        hw = jnp.dot(x_ref[...], w1_ref[...], preferred_element_type=jnp.float32)
        u1_ref[0:n, 0:h1] = hw * dinv[0:n, :]
        u1_ref[n:np_pad, 0:h1] = jnp.zeros((np_pad - n, h1), jnp.float32)
        u1_ref[:, h1:16] = jnp.zeros((np_pad, 16 - h1), jnp.float32)

    u1, dinv = _tc_call(tc1, [(np_pad, 16), (np_pad, 1)], x, W1, degp2)

    accp1 = _sc_prop16(src3, dst3, u1, np_pad, epw)

    def tc2(accp_ref, u1_ref, dinv_ref, b1_ref, w2_ref, u2_ref):
        acc = (accp_ref[0:np_pad, 0:h1] + accp_ref[np_pad:2 * np_pad, 0:h1]
               - u1_ref[:, 0:h1])
        hh = jnp.maximum(dinv_ref[0:n, :] * acc[0:n, :] + b1_ref[...], 0.0)
        u2_ref[0:n, 0:h2] = jnp.dot(hh, w2_ref[...],
                                    preferred_element_type=jnp.float32) * dinv_ref[0:n, :]
        u2_ref[n:np_pad, 0:h2] = jnp.zeros((np_pad - n, h2), jnp.float32)
        u2_ref[:, h2:16] = jnp.zeros((np_pad, 16 - h2), jnp.float32)

    u2, = _tc_call(tc2, [(np_pad, 16)], accp1, u1, dinv, b1.reshape(1, h1), W2)

    accp2 = _sc_prop16(src3, dst3, u2, np_pad, epw)

    def tc3(accp_ref, u2_ref, dinv_ref, b2_ref, wc_ref, bc_ref, out_ref, h_ref):
        acc = (accp_ref[0:np_pad, 0:h2] + accp_ref[np_pad:2 * np_pad, 0:h2]
               - u2_ref[:, 0:h2])
        hh = jnp.tanh(dinv_ref[0:n, :] * acc[0:n, :] + b2_ref[...])
        h_ref[...] = hh
        z = jnp.dot(hh, wc_ref[...], preferred_element_type=jnp.float32) + bc_ref[...]
        out_ref[...] = jax.nn.sigmoid(z)

    out, h = _tc_call(tc3, [(n, 1), (n, h2)], accp2, u2, dinv,
                      b2.reshape(1, h2), Wc, bc.reshape(1, 1))
    return (out, h)
